# baseline (speedup 1.0000x reference)
import functools

import jax
import jax.numpy as jnp
from jax import lax
from jax.experimental import pallas as pl
from jax.experimental.pallas import tpu as pltpu
from jax.experimental.pallas import tpu_sc as plsc

EMBED_DIM = 64
NUM_WORKERS = 32
K = 2
NBUF = 4


@functools.partial(jax.jit, static_argnames=("n_chunks", "seq_len"))
def _gather(weight, idx, n_chunks, seq_len):
    n_rows = idx.shape[0] * idx.shape[1] * K // 1
    n_x_rows = NUM_WORKERS * n_chunks * K
    chunk_idx = K * seq_len
    mesh = plsc.VectorSubcoreMesh(core_axis_name="c", subcore_axis_name="s")
    n_groups = n_chunks // NBUF

    @functools.partial(
        pl.kernel,
        mesh=mesh,
        out_type=jax.ShapeDtypeStruct((n_x_rows, seq_len, EMBED_DIM),
                                      jnp.float32),
        scratch_types=[
            pltpu.VMEM((n_chunks, chunk_idx), jnp.int32),
            [pltpu.VMEM((chunk_idx, EMBED_DIM), jnp.float32)
             for _ in range(NBUF)],
            [pltpu.SemaphoreType.DMA for _ in range(NBUF)],
            [pltpu.SemaphoreType.DMA for _ in range(NBUF)],
        ],
        compiler_params=pltpu.CompilerParams(use_tc_tiling_on_sc=False),
    )
    def body(weight_hbm, idx_hbm, out_hbm, idx_v, rows, gsems, wsems):
        wid = lax.axis_index("s") * 2 + lax.axis_index("c")
        xrow_base = wid * (n_chunks * K)
        pltpu.sync_copy(idx_hbm.at[wid], idx_v)

        def gather_copy(j, b):
            return pltpu.make_async_copy(
                weight_hbm.at[idx_v.at[j]], rows[b], gsems[b])

        def write_copies(j, b):
            return [
                pltpu.make_async_copy(
                    rows[b].at[pl.ds(r * seq_len, seq_len)],
                    out_hbm.at[xrow_base + j * K + r],
                    wsems[b])
                for r in range(K)
            ]

        for b in range(NBUF):
            gather_copy(b, b).start()

        def group(g, carry):
            j0 = g * NBUF
            for b in range(NBUF):
                gather_copy(j0 + b, b).wait()
                for w in write_copies(j0 + b, b):
                    w.start()
            @pl.when(g + 1 < n_groups)
            def _():
                for b in range(NBUF):
                    for w in write_copies(j0 + b, b):
                        w.wait()
                    gather_copy(j0 + NBUF + b, b).start()
            return carry

        lax.fori_loop(0, n_groups, group, 0)

        for b in range(NBUF):
            for w in write_copies((n_groups - 1) * NBUF + b, b):
                w.wait()

    return body(weight, idx)


def kernel(x, weight):
    n_x_rows, seq_len = x.shape
    rows_per_w = n_x_rows // NUM_WORKERS
    n_chunks = rows_per_w // K
    idx = x.reshape(NUM_WORKERS, n_chunks, K * seq_len).astype(jnp.int32)
    return _gather(weight, idx, n_chunks, seq_len)
